# Initial kernel scaffold; baseline (speedup 1.0000x reference)
import jax
import jax.numpy as jnp
from jax.experimental import pallas as pl


def kernel(x, edge_index, W_x, b_x, W_y, b_y, W_th, b_th, W_v, b_v, W_e1, b_e1, W_x2, b_x2, W_e2, b_e2, W_out, b_out):
    raise NotImplementedError("write your pallas kernel here")


# speedup vs baseline: 16.5521x; 16.5521x over previous
import functools

import jax
import jax.numpy as jnp
from jax import lax
from jax.experimental import pallas as pl
from jax.experimental.pallas import tpu as pltpu
from jax.experimental.pallas import tpu_sc as plsc

N = 50000
E = 800000
L = 16
NC = 2
NS = 16
NW = NC * NS
N_PAD = 51200
ROWS_PER_TILE = N_PAD // NS
EP_BLOCKS = 6400
EP = EP_BLOCKS * 128
BLOCKS_PER_TILE = EP_BLOCKS // NW
G = 8
RB = 1024
N_BLOCKS = N_PAD // RB


def _seg16_body(table, src_r, dst_r, out, src_v, dst_v, rows_v, acc, sem):
    cid = lax.axis_index("c")
    sid = lax.axis_index("s")
    wid = cid * NS + sid

    zero = jnp.zeros((L,), jnp.float32)

    def zrows(i, carry):
        rows_v[i] = zero
        return carry

    lax.fori_loop(0, 128, zrows, 0)

    def zacc(k, carry):
        pltpu.sync_copy(rows_v, acc.at[pl.ds(sid * ROWS_PER_TILE + k * 128, 128)])
        return carry

    lax.fori_loop(0, ROWS_PER_TILE // 128, zacc, 0)
    plsc.subcore_barrier()

    def step(g, carry):
        base = wid * BLOCKS_PER_TILE + g * G
        pltpu.sync_copy(src_r.at[pl.ds(base, G)], src_v)
        pltpu.sync_copy(dst_r.at[pl.ds(base, G)], dst_v)
        for j in range(G):
            pltpu.async_copy(table.at[src_v.at[j]], rows_v, sem).wait()
            pltpu.sync_copy(rows_v, acc.at[dst_v.at[j]], add=True)
        return carry

    lax.fori_loop(0, BLOCKS_PER_TILE // G, step, 0)
    plsc.subcore_barrier()
    pltpu.sync_copy(
        acc.at[pl.ds(sid * ROWS_PER_TILE, ROWS_PER_TILE)],
        out.at[cid, pl.ds(sid * ROWS_PER_TILE, ROWS_PER_TILE)],
    )


def _seg16(table, src_r, dst_r):
    run = functools.partial(
        pl.kernel,
        mesh=plsc.VectorSubcoreMesh(core_axis_name="c", subcore_axis_name="s"),
        out_type=jax.ShapeDtypeStruct((NC, N_PAD, L), jnp.float32),
        compiler_params=pltpu.CompilerParams(use_tc_tiling_on_sc=False),
        scratch_types=[
            pltpu.VMEM((G, 128), jnp.int32),
            pltpu.VMEM((G, 128), jnp.int32),
            pltpu.VMEM((128, L), jnp.float32),
            pltpu.VMEM_SHARED((N_PAD, L), jnp.float32),
            pltpu.SemaphoreType.DMA,
        ],
    )(_seg16_body)
    return run(table, src_r, dst_r)


def _u_body(coef_ref, xa_ref, p1a_ref, p1b_ref, out_ref):
    s = p1a_ref[0] + p1b_ref[0]
    xa = xa_ref[...]
    b1 = coef_ref[0:1, :]
    b4 = coef_ref[1:2, :]
    c1 = coef_ref[2:3, 0:1]
    c = s[:, 10:11]
    inv = 1.0 / jnp.maximum(c, 1.0)
    has = (c >= 0.5).astype(jnp.float32)
    u = (has * jnp.sum(xa * b1, axis=1, keepdims=True)
         + inv * jnp.sum(s * b4, axis=1, keepdims=True) + c1)
    out_ref[...] = jnp.broadcast_to(u, (RB, L))


def _out_body(coef_ref, xa_ref, p1a_ref, p1b_ref, p2a_ref, p2b_ref, out_ref):
    s = p1a_ref[0] + p1b_ref[0]
    w = p2a_ref[0][:, 0:1] + p2b_ref[0][:, 0:1]
    xa = xa_ref[...]
    a1 = coef_ref[0:1, :]
    a4 = coef_ref[1:2, :]
    c0 = coef_ref[2:3, 0:1]
    bo = coef_ref[2:3, 1:2]
    c = s[:, 10:11]
    inv = 1.0 / jnp.maximum(c, 1.0)
    has = (c >= 0.5).astype(jnp.float32)
    val = (has * jnp.sum(xa * a1, axis=1, keepdims=True)
           + inv * jnp.sum(s * a4, axis=1, keepdims=True)
           + has * c0 + inv * w + bo)
    out_ref[...] = jnp.broadcast_to(val, (RB, L))


def _node_stage(body, n_extra):
    full = pl.BlockSpec((4, L), lambda i: (0, 0))
    rows = pl.BlockSpec((RB, L), lambda i: (i, 0))
    part0 = pl.BlockSpec((1, RB, L), lambda i: (0, i, 0))
    part1 = pl.BlockSpec((1, RB, L), lambda i: (1, i, 0))
    in_specs = [full, rows] + [part0, part1] * n_extra
    return pl.pallas_call(
        body,
        grid=(N_BLOCKS,),
        in_specs=in_specs,
        out_specs=rows,
        out_shape=jax.ShapeDtypeStruct((N_PAD, L), jnp.float32),
    )


def kernel(x, edge_index, W_x, b_x, W_y, b_y, W_th, b_th, W_v, b_v,
           W_e1, b_e1, W_x2, b_x2, W_e2, b_e2, W_out, b_out):
    f32 = jnp.float32

    M = jnp.concatenate(
        [W_x @ W_v[0:64], W_y @ W_v[64:128], W_th @ W_v[128:192]], axis=0)
    m0 = b_x @ W_v[0:64] + b_y @ W_v[64:128] + b_th @ W_v[128:192] + b_v
    A = W_e1[0:64]
    B = W_e1[64:128]
    C = W_e1[128:130]
    D = W_e1[130:132]
    p = (W_e2[0:64] @ W_out)[:, 0]
    q = (W_e2[64:128] @ W_out)[:, 0]
    rp = W_x2 @ p
    rq = W_x2 @ q

    def fold(r):
        ar = A @ r
        br = B @ r
        zeros5 = jnp.zeros((5,), f32)
        dst_c = jnp.concatenate([M @ ar, C @ r, (m0 @ ar + b_e1 @ r)[None], zeros5])
        src_c = jnp.concatenate([M @ br, D @ r, (m0 @ br)[None], zeros5])
        return dst_c, src_c

    a1e, a4e = fold(rp)
    b1e, b4e = fold(rq)
    c0 = b_x2 @ p + (b_e2 @ W_out)[0]
    c1 = b_x2 @ q
    bo = b_out[0]

    ucoef = jnp.zeros((4, L), f32).at[0].set(b1e).at[1].set(b4e) \
        .at[2, 0].set(c1)
    ocoef = jnp.zeros((4, L), f32).at[0].set(a1e).at[1].set(a4e) \
        .at[2, 0].set(c0).at[2, 1].set(bo)

    src = edge_index[0].astype(jnp.int32)
    dst = edge_index[1].astype(jnp.int32)
    pad_idx = jnp.full((EP - E,), N, jnp.int32)
    src_r = jnp.concatenate([src, pad_idx]).reshape(EP_BLOCKS, 128)
    dst_r = jnp.concatenate([dst, pad_idx]).reshape(EP_BLOCKS, 128)
    xa = jnp.zeros((N_PAD, L), f32)
    xa = xa.at[:N, 0:10].set(x).at[:N, 10].set(1.0)

    p1 = _seg16(xa, src_r, dst_r)

    u16 = _node_stage(_u_body, 1)(ucoef, xa, p1, p1)

    p2 = _seg16(u16, src_r, dst_r)

    o16 = _node_stage(_out_body, 2)(ocoef, xa, p1, p1, p2, p2)
    return o16[:N, 0].reshape(1, N)


# speedup vs baseline: 27.5863x; 1.6666x over previous
import functools

import jax
import jax.numpy as jnp
from jax import lax
from jax.experimental import pallas as pl
from jax.experimental.pallas import tpu as pltpu
from jax.experimental.pallas import tpu_sc as plsc

N = 50000
E = 800000
L = 16
NC = 2
NS = 16
NW = NC * NS
N_PAD = 51200
ROWS_PER_TILE = N_PAD // NS
EB = 100
E_BLOCKS = E // EB
BLOCKS_PER_TILE = E_BLOCKS // NW
G = 10
STEPS = BLOCKS_PER_TILE // G
RB = 1024
N_BLOCKS = N_PAD // RB


def _seg16_body(table, src_r, dst_r, out, src_v, dst_v, rows_v, acc, gsem, ssem):
    cid = lax.axis_index("c")
    sid = lax.axis_index("s")
    wid = cid * NS + sid

    zero = jnp.zeros((L,), jnp.float32)

    def zrows(i, carry):
        rows_v[i] = zero
        return carry

    lax.fori_loop(0, 128, zrows, 0)

    def zacc(k, carry):
        pltpu.sync_copy(
            rows_v.at[pl.ds(0, 128)],
            acc.at[pl.ds(sid * ROWS_PER_TILE + k * 128, 128)])
        return carry

    lax.fori_loop(0, ROWS_PER_TILE // 128, zacc, 0)
    plsc.subcore_barrier()

    def step(g, carry):
        base = wid * BLOCKS_PER_TILE + g * G
        pltpu.sync_copy(src_r.at[pl.ds(base, G)], src_v)
        pltpu.sync_copy(dst_r.at[pl.ds(base, G)], dst_v)
        gds = [
            pltpu.async_copy(
                table.at[src_v.at[j]], rows_v.at[pl.ds(j * EB, EB)], gsem)
            for j in range(G)
        ]
        for d in gds:
            d.wait()
        sds = [
            pltpu.async_copy(
                rows_v.at[pl.ds(j * EB, EB)], acc.at[dst_v.at[j]], ssem,
                add=True)
            for j in range(G)
        ]
        for d in sds:
            d.wait()
        return carry

    lax.fori_loop(0, STEPS, step, 0)
    plsc.subcore_barrier()
    pltpu.sync_copy(
        acc.at[pl.ds(sid * ROWS_PER_TILE, ROWS_PER_TILE)],
        out.at[cid, pl.ds(sid * ROWS_PER_TILE, ROWS_PER_TILE)],
    )


def _seg16(table, src_r, dst_r):
    run = functools.partial(
        pl.kernel,
        mesh=plsc.VectorSubcoreMesh(core_axis_name="c", subcore_axis_name="s"),
        out_type=jax.ShapeDtypeStruct((NC, N_PAD, L), jnp.float32),
        compiler_params=pltpu.CompilerParams(use_tc_tiling_on_sc=False),
        scratch_types=[
            pltpu.VMEM((G, EB), jnp.int32),
            pltpu.VMEM((G, EB), jnp.int32),
            pltpu.VMEM((G * EB, L), jnp.float32),
            pltpu.VMEM_SHARED((N_PAD, L), jnp.float32),
            pltpu.SemaphoreType.DMA,
            pltpu.SemaphoreType.DMA,
        ],
    )(_seg16_body)
    return run(table, src_r, dst_r)


def _u_body(coef_ref, xa_ref, p1a_ref, p1b_ref, out_ref):
    s = p1a_ref[0] + p1b_ref[0]
    xa = xa_ref[...]
    b1 = coef_ref[0:1, :]
    b4 = coef_ref[1:2, :]
    c1 = coef_ref[2:3, 0:1]
    c = s[:, 10:11]
    inv = 1.0 / jnp.maximum(c, 1.0)
    has = (c >= 0.5).astype(jnp.float32)
    u = (has * jnp.sum(xa * b1, axis=1, keepdims=True)
         + inv * jnp.sum(s * b4, axis=1, keepdims=True) + c1)
    out_ref[...] = jnp.broadcast_to(u, (RB, L))


def _out_body(coef_ref, xa_ref, p1a_ref, p1b_ref, p2a_ref, p2b_ref, out_ref):
    s = p1a_ref[0] + p1b_ref[0]
    w = p2a_ref[0][:, 0:1] + p2b_ref[0][:, 0:1]
    xa = xa_ref[...]
    a1 = coef_ref[0:1, :]
    a4 = coef_ref[1:2, :]
    c0 = coef_ref[2:3, 0:1]
    bo = coef_ref[2:3, 1:2]
    c = s[:, 10:11]
    inv = 1.0 / jnp.maximum(c, 1.0)
    has = (c >= 0.5).astype(jnp.float32)
    val = (has * jnp.sum(xa * a1, axis=1, keepdims=True)
           + inv * jnp.sum(s * a4, axis=1, keepdims=True)
           + has * c0 + inv * w + bo)
    out_ref[...] = jnp.broadcast_to(val, (RB, L))


def _node_stage(body, n_extra):
    full = pl.BlockSpec((4, L), lambda i: (0, 0))
    rows = pl.BlockSpec((RB, L), lambda i: (i, 0))
    part0 = pl.BlockSpec((1, RB, L), lambda i: (0, i, 0))
    part1 = pl.BlockSpec((1, RB, L), lambda i: (1, i, 0))
    in_specs = [full, rows] + [part0, part1] * n_extra
    return pl.pallas_call(
        body,
        grid=(N_BLOCKS,),
        in_specs=in_specs,
        out_specs=rows,
        out_shape=jax.ShapeDtypeStruct((N_PAD, L), jnp.float32),
    )


def kernel(x, edge_index, W_x, b_x, W_y, b_y, W_th, b_th, W_v, b_v,
           W_e1, b_e1, W_x2, b_x2, W_e2, b_e2, W_out, b_out):
    f32 = jnp.float32

    M = jnp.concatenate(
        [W_x @ W_v[0:64], W_y @ W_v[64:128], W_th @ W_v[128:192]], axis=0)
    m0 = b_x @ W_v[0:64] + b_y @ W_v[64:128] + b_th @ W_v[128:192] + b_v
    A = W_e1[0:64]
    B = W_e1[64:128]
    C = W_e1[128:130]
    D = W_e1[130:132]
    p = (W_e2[0:64] @ W_out)[:, 0]
    q = (W_e2[64:128] @ W_out)[:, 0]
    rp = W_x2 @ p
    rq = W_x2 @ q

    def fold(r):
        ar = A @ r
        br = B @ r
        zeros5 = jnp.zeros((5,), f32)
        dst_c = jnp.concatenate([M @ ar, C @ r, (m0 @ ar + b_e1 @ r)[None], zeros5])
        src_c = jnp.concatenate([M @ br, D @ r, (m0 @ br)[None], zeros5])
        return dst_c, src_c

    a1e, a4e = fold(rp)
    b1e, b4e = fold(rq)
    c0 = b_x2 @ p + (b_e2 @ W_out)[0]
    c1 = b_x2 @ q
    bo = b_out[0]

    ucoef = jnp.zeros((4, L), f32).at[0].set(b1e).at[1].set(b4e) \
        .at[2, 0].set(c1)
    ocoef = jnp.zeros((4, L), f32).at[0].set(a1e).at[1].set(a4e) \
        .at[2, 0].set(c0).at[2, 1].set(bo)

    src_r = edge_index[0].astype(jnp.int32).reshape(E_BLOCKS, EB)
    dst_r = edge_index[1].astype(jnp.int32).reshape(E_BLOCKS, EB)
    xa = jnp.pad(
        jnp.concatenate([x, jnp.ones((N, 1), f32)], axis=1),
        ((0, N_PAD - N), (0, L - 11)))

    p1 = _seg16(xa, src_r, dst_r)

    u16 = _node_stage(_u_body, 1)(ucoef, xa, p1, p1)

    p2 = _seg16(u16, src_r, dst_r)

    o16 = _node_stage(_out_body, 2)(ocoef, xa, p1, p1, p2, p2)
    return o16[:N, 0].reshape(1, N)


# speedup vs baseline: 39.4586x; 1.4304x over previous
import functools

import jax
import jax.numpy as jnp
from jax import lax
from jax.experimental import pallas as pl
from jax.experimental.pallas import tpu as pltpu
from jax.experimental.pallas import tpu_sc as plsc

N = 50000
E = 800000
F = 8
NC = 2
NS = 16
NW = NC * NS
N_PAD = 51200
ROWS_PER_TILE = N_PAD // NS
EB = 100
E_BLOCKS = E // EB
BLOCKS_PER_TILE = E_BLOCKS // NW
G = 10
STEPS = BLOCKS_PER_TILE // G
WR = N_PAD * F // 128
RBW = 320
WB = WR // RBW


def _seg8_body(table, edges, zrow, out, src_v, dst_v, rows_v, acc, gsem, ssem):
    cid = lax.axis_index("c")
    sid = lax.axis_index("s")
    wid = cid * NS + sid

    def zacc(k, carry):
        pltpu.sync_copy(
            zrow, acc.at[pl.ds(sid * ROWS_PER_TILE + k * 400, 400)])
        return carry

    lax.fori_loop(0, ROWS_PER_TILE // 400, zacc, 0)
    plsc.subcore_barrier()

    def step(g, carry):
        base = wid * BLOCKS_PER_TILE + g * G
        pltpu.sync_copy(edges.at[0, pl.ds(base, G)], src_v)
        pltpu.sync_copy(edges.at[1, pl.ds(base, G)], dst_v)
        gds = [
            pltpu.async_copy(
                table.at[src_v.at[j]], rows_v.at[pl.ds(j * EB, EB)], gsem)
            for j in range(G)
        ]
        for d in gds:
            d.wait()
        sds = [
            pltpu.async_copy(
                rows_v.at[pl.ds(j * EB, EB)], acc.at[dst_v.at[j]], ssem,
                add=True)
            for j in range(G)
        ]
        for d in sds:
            d.wait()
        return carry

    lax.fori_loop(0, STEPS, step, 0)
    plsc.subcore_barrier()
    pltpu.sync_copy(
        acc.at[pl.ds(sid * ROWS_PER_TILE, ROWS_PER_TILE)],
        out.at[cid, pl.ds(sid * ROWS_PER_TILE, ROWS_PER_TILE)],
    )


def _seg8(table, edges, zrow):
    run = functools.partial(
        pl.kernel,
        mesh=plsc.VectorSubcoreMesh(core_axis_name="c", subcore_axis_name="s"),
        out_type=jax.ShapeDtypeStruct((NC, N_PAD, F), jnp.float32),
        compiler_params=pltpu.CompilerParams(use_tc_tiling_on_sc=False),
        scratch_types=[
            pltpu.VMEM((G, EB), jnp.int32),
            pltpu.VMEM((G, EB), jnp.int32),
            pltpu.VMEM((G * EB, F), jnp.float32),
            pltpu.VMEM_SHARED((N_PAD, F), jnp.float32),
            pltpu.SemaphoreType.DMA,
            pltpu.SemaphoreType.DMA,
        ],
    )(_seg8_body)
    return run(table, edges, zrow)


def _tab_body(t2_ref, xw_ref, out_ref):
    out_ref[...] = jnp.dot(xw_ref[...], t2_ref[...],
                           preferred_element_type=jnp.float32)


def _u_body(bm_ref, cc_ref, tab_ref, p1a_ref, p1b_ref, out_ref):
    s = p1a_ref[0] + p1b_ref[0]
    bc = bm_ref[0]
    b0 = bm_ref[1]
    b3 = bm_ref[2]
    c1 = cc_ref[0:1, 0:1]
    c = jnp.dot(s, bc, preferred_element_type=jnp.float32)
    inv = 1.0 / jnp.maximum(c, 1.0)
    has = (c >= 0.5).astype(jnp.float32)
    g1 = jnp.dot(s, b0, preferred_element_type=jnp.float32)
    d1 = jnp.dot(tab_ref[...], b3, preferred_element_type=jnp.float32)
    out_ref[...] = has * d1 + inv * g1 + c1


def _out_body(bm_ref, cc_ref, tab_ref, p1a_ref, p1b_ref, p2a_ref, p2b_ref,
              out_ref):
    s = p1a_ref[0] + p1b_ref[0]
    p2 = p2a_ref[0] + p2b_ref[0]
    bc = bm_ref[0]
    b0 = bm_ref[1]
    b1 = bm_ref[3]
    b4 = bm_ref[4]
    c0 = cc_ref[0:1, 1:2]
    bo = cc_ref[0:1, 2:3]
    c = jnp.dot(s, bc, preferred_element_type=jnp.float32)
    inv = 1.0 / jnp.maximum(c, 1.0)
    has = (c >= 0.5).astype(jnp.float32)
    g2 = jnp.dot(s, b1, preferred_element_type=jnp.float32)
    d2 = jnp.dot(tab_ref[...], b4, preferred_element_type=jnp.float32)
    w = jnp.dot(p2, b0, preferred_element_type=jnp.float32)
    out_ref[...] = has * d2 + inv * g2 + has * c0 + inv * w + bo


_WIDE = jax.ShapeDtypeStruct((WR, 128), jnp.float32)


def _tab_stage(t2, xw2):
    return pl.pallas_call(
        _tab_body,
        grid=(WB,),
        in_specs=[pl.BlockSpec((256, 128), lambda i: (0, 0)),
                  pl.BlockSpec((RBW, 256), lambda i: (i, 0))],
        out_specs=pl.BlockSpec((RBW, 128), lambda i: (i, 0)),
        out_shape=_WIDE,
    )(t2, xw2)


def _u_stage(bm, cc, tab_w, p1w):
    rows = pl.BlockSpec((RBW, 128), lambda i: (i, 0))
    return pl.pallas_call(
        _u_body,
        grid=(WB,),
        in_specs=[pl.BlockSpec((5, 128, 128), lambda i: (0, 0, 0)),
                  pl.BlockSpec((1, 128), lambda i: (0, 0)),
                  rows,
                  pl.BlockSpec((1, RBW, 128), lambda i: (0, i, 0)),
                  pl.BlockSpec((1, RBW, 128), lambda i: (1, i, 0))],
        out_specs=rows,
        out_shape=_WIDE,
    )(bm, cc, tab_w, p1w, p1w)


def _out_stage(bm, cc, tab_w, p1w, p2w):
    rows = pl.BlockSpec((RBW, 128), lambda i: (i, 0))
    part0 = pl.BlockSpec((1, RBW, 128), lambda i: (0, i, 0))
    part1 = pl.BlockSpec((1, RBW, 128), lambda i: (1, i, 0))
    return pl.pallas_call(
        _out_body,
        grid=(WB,),
        in_specs=[pl.BlockSpec((5, 128, 128), lambda i: (0, 0, 0)),
                  pl.BlockSpec((1, 128), lambda i: (0, 0)),
                  rows, part0, part1, part0, part1],
        out_specs=rows,
        out_shape=_WIDE,
    )(bm, cc, tab_w, p1w, p1w, p2w, p2w)


def kernel(x, edge_index, W_x, b_x, W_y, b_y, W_th, b_th, W_v, b_v,
           W_e1, b_e1, W_x2, b_x2, W_e2, b_e2, W_out, b_out):
    f32 = jnp.float32

    M = jnp.concatenate(
        [W_x @ W_v[0:64], W_y @ W_v[64:128], W_th @ W_v[128:192]], axis=0)
    m0 = b_x @ W_v[0:64] + b_y @ W_v[64:128] + b_th @ W_v[128:192] + b_v
    A = W_e1[0:64]
    B = W_e1[64:128]
    C = W_e1[128:130]
    D = W_e1[130:132]
    p = (W_e2[0:64] @ W_out)[:, 0]
    q = (W_e2[64:128] @ W_out)[:, 0]
    rp = W_x2 @ p
    rq = W_x2 @ q

    def fold(r):
        ar = A @ r
        br = B @ r
        zeros5 = jnp.zeros((5,), f32)
        dst_c = jnp.concatenate([M @ ar, C @ r, (m0 @ ar + b_e1 @ r)[None], zeros5])
        src_c = jnp.concatenate([M @ br, D @ r, (m0 @ br)[None], zeros5])
        return dst_c, src_c

    a1e, a4e = fold(rp)
    b1e, b4e = fold(rq)
    c0 = b_x2 @ p + (b_e2 @ W_out)[0]
    c1 = b_x2 @ q
    bo = b_out[0]

    ones16 = jnp.zeros((16,), f32).at[10].set(1.0)
    cf8 = jnp.stack([b4e, a4e, ones16, b1e, a1e,
                     jnp.zeros(16, f32), jnp.zeros(16, f32),
                     jnp.zeros(16, f32)], axis=1)

    a_idx = jnp.arange(256)
    b_idx = jnp.arange(128)
    same_node = (a_idx[:, None] // 16) == (b_idx[None, :] // F)
    t2 = jnp.where(same_node, cf8[a_idx % 16][:, b_idx % F], 0.0)

    same_grp = (b_idx[:, None] // F) == (b_idx[None, :] // F)
    bms = jnp.stack([
        jnp.where(same_grp & ((b_idx[:, None] % F) == f), 1.0, 0.0)
        for f in (2, 0, 3, 1, 4)], axis=0)
    cc = jnp.zeros((1, 128), f32).at[0, 0].set(c1).at[0, 1].set(c0) \
        .at[0, 2].set(bo)

    edges = edge_index.astype(jnp.int32).reshape(2, E_BLOCKS, EB)
    xa16 = jnp.pad(jnp.concatenate([x, jnp.ones((N, 1), f32)], axis=1),
                   ((0, N_PAD - N), (0, 5)))
    xw2 = jnp.reshape(xa16, (WR, 256))

    tab_w = _tab_stage(t2, xw2)
    tab8 = jnp.reshape(tab_w, (N_PAD, F))

    zrow = jnp.zeros((400, F), f32)

    p1 = _seg8(tab8, edges, zrow)
    p1w = jnp.reshape(p1, (NC, WR, 128))

    u_w = _u_stage(bms, cc, tab_w, p1w)
    u8 = jnp.reshape(u_w, (N_PAD, F))

    p2 = _seg8(u8, edges, zrow)
    p2w = jnp.reshape(p2, (NC, WR, 128))

    o_w = _out_stage(bms, cc, tab_w, p1w, p2w)
    return jnp.reshape(o_w, (N_PAD, F))[:N, 0].reshape(1, N)


# speedup vs baseline: 51.1906x; 1.2973x over previous
import functools

import jax
import jax.numpy as jnp
from jax import lax
from jax.experimental import pallas as pl
from jax.experimental.pallas import tpu as pltpu
from jax.experimental.pallas import tpu_sc as plsc

N = 50000
E = 800000
F = 8
NC = 2
NS = 16
NW = NC * NS
N_PAD = 51200
ROWS_PER_TILE = N_PAD // NS
EB = 100
E_BLOCKS = E // EB
BLOCKS_PER_TILE = E_BLOCKS // NW
G = 10
STEPS = BLOCKS_PER_TILE // G
WR = N_PAD * F // 128
RBW = 320
WB = WR // RBW
XB = 5120
CR = N_PAD // 128


def _seg8_body(table, edges, zrow, out, src_v, dst_v, rows_v, acc, gsem, ssem):
    cid = lax.axis_index("c")
    sid = lax.axis_index("s")
    wid = cid * NS + sid

    def zacc(k, carry):
        pltpu.sync_copy(
            zrow, acc.at[pl.ds(sid * ROWS_PER_TILE + k * 400, 400)])
        return carry

    lax.fori_loop(0, ROWS_PER_TILE // 400, zacc, 0)
    plsc.subcore_barrier()

    def load_idx(slot, g):
        base = wid * BLOCKS_PER_TILE + g * G
        pltpu.sync_copy(edges.at[0, pl.ds(base, G)], src_v.at[slot])
        pltpu.sync_copy(edges.at[1, pl.ds(base, G)], dst_v.at[slot])

    def fire_gathers(slot):
        return [
            pltpu.async_copy(
                table.at[src_v.at[slot].at[j]],
                rows_v.at[slot].at[pl.ds(j * EB, EB)], gsem)
            for j in range(G)
        ]

    def drain_gathers(slot):
        for j in range(G):
            pltpu.make_async_copy(
                table.at[src_v.at[slot].at[j]],
                rows_v.at[slot].at[pl.ds(j * EB, EB)], gsem).wait()

    def fire_scatters(slot):
        return [
            pltpu.async_copy(
                rows_v.at[slot].at[pl.ds(j * EB, EB)],
                acc.at[dst_v.at[slot].at[j]], ssem, add=True)
            for j in range(G)
        ]

    def drain_scatters(slot):
        for j in range(G):
            pltpu.make_async_copy(
                rows_v.at[slot].at[pl.ds(j * EB, EB)],
                acc.at[dst_v.at[slot].at[j]], ssem).wait()

    load_idx(0, 0)
    fire_gathers(0)

    def step(g, carry):
        s = lax.rem(g, 2)
        n = 1 - s

        @pl.when(g + 1 < STEPS)
        def _prefetch():
            load_idx(n, g + 1)

        drain_gathers(s)

        @pl.when(g >= 1)
        def _drain_prev():
            drain_scatters(n)

        fire_scatters(s)

        @pl.when(g + 1 < STEPS)
        def _fire_next():
            fire_gathers(n)

        return carry

    lax.fori_loop(0, STEPS, step, 0)
    drain_scatters((STEPS - 1) % 2)
    plsc.subcore_barrier()
    pltpu.sync_copy(
        acc.at[pl.ds(sid * ROWS_PER_TILE, ROWS_PER_TILE)],
        out.at[cid, pl.ds(sid * ROWS_PER_TILE, ROWS_PER_TILE)],
    )


def _seg8(table, edges, zrow):
    run = functools.partial(
        pl.kernel,
        mesh=plsc.VectorSubcoreMesh(core_axis_name="c", subcore_axis_name="s"),
        out_type=jax.ShapeDtypeStruct((NC, N_PAD, F), jnp.float32),
        compiler_params=pltpu.CompilerParams(use_tc_tiling_on_sc=False),
        scratch_types=[
            pltpu.VMEM((2, G, EB), jnp.int32),
            pltpu.VMEM((2, G, EB), jnp.int32),
            pltpu.VMEM((2, G * EB, F), jnp.float32),
            pltpu.VMEM_SHARED((N_PAD, F), jnp.float32),
            pltpu.SemaphoreType.DMA,
            pltpu.SemaphoreType.DMA,
        ],
    )(_seg8_body)
    return run(table, edges, zrow)


def _tab_body(t2_ref, xw_ref, out_ref):
    out_ref[...] = jnp.dot(xw_ref[...], t2_ref[...],
                           preferred_element_type=jnp.float32)


def _u_body(bm_ref, cc_ref, tab_ref, p1a_ref, p1b_ref, out_ref):
    s = p1a_ref[0] + p1b_ref[0]
    bc = bm_ref[0]
    b0 = bm_ref[1]
    b3 = bm_ref[2]
    c1 = cc_ref[0:1, 0:1]
    c = jnp.dot(s, bc, preferred_element_type=jnp.float32)
    inv = 1.0 / jnp.maximum(c, 1.0)
    has = (c >= 0.5).astype(jnp.float32)
    g1 = jnp.dot(s, b0, preferred_element_type=jnp.float32)
    d1 = jnp.dot(tab_ref[...], b3, preferred_element_type=jnp.float32)
    out_ref[...] = has * d1 + inv * g1 + c1


def _out_body(bm_ref, cc_ref, mf_ref, pj_ref, tab_ref, p1a_ref, p1b_ref,
              p2a_ref, p2b_ref, out_ref):
    s = p1a_ref[0] + p1b_ref[0]
    p2 = p2a_ref[0] + p2b_ref[0]
    bc = bm_ref[0]
    b0 = bm_ref[1]
    b1 = bm_ref[3]
    b4 = bm_ref[4]
    c0 = cc_ref[0:1, 1:2]
    bo = cc_ref[0:1, 2:3]
    c = jnp.dot(s, bc, preferred_element_type=jnp.float32)
    inv = 1.0 / jnp.maximum(c, 1.0)
    has = (c >= 0.5).astype(jnp.float32)
    g2 = jnp.dot(s, b1, preferred_element_type=jnp.float32)
    d2 = jnp.dot(tab_ref[...], b4, preferred_element_type=jnp.float32)
    w = jnp.dot(p2, b0, preferred_element_type=jnp.float32)
    val = has * d2 + inv * g2 + has * c0 + inv * w + bo
    acc = jnp.zeros((RBW // 8, 128), jnp.float32)
    for j in range(8):
        acc = acc + jnp.dot(
            pj_ref[j],
            jnp.dot(val, mf_ref[j], preferred_element_type=jnp.float32),
            preferred_element_type=jnp.float32)
    out_ref[...] = acc


_WIDE = jax.ShapeDtypeStruct((WR, 128), jnp.float32)


def _tab_stage(t2, xw2):
    return pl.pallas_call(
        _tab_body,
        grid=(WB,),
        in_specs=[pl.BlockSpec((256, 128), lambda i: (0, 0)),
                  pl.BlockSpec((RBW, 256), lambda i: (i, 0))],
        out_specs=pl.BlockSpec((RBW, 128), lambda i: (i, 0)),
        out_shape=_WIDE,
    )(t2, xw2)


def _u_stage(bm, cc, tab_w, p1w):
    rows = pl.BlockSpec((RBW, 128), lambda i: (i, 0))
    return pl.pallas_call(
        _u_body,
        grid=(WB,),
        in_specs=[pl.BlockSpec((5, 128, 128), lambda i: (0, 0, 0)),
                  pl.BlockSpec((1, 128), lambda i: (0, 0)),
                  rows,
                  pl.BlockSpec((1, RBW, 128), lambda i: (0, i, 0)),
                  pl.BlockSpec((1, RBW, 128), lambda i: (1, i, 0))],
        out_specs=rows,
        out_shape=_WIDE,
    )(bm, cc, tab_w, p1w, p1w)


def _out_stage(bm, cc, mf, pj, tab_w, p1w, p2w):
    rows = pl.BlockSpec((RBW, 128), lambda i: (i, 0))
    part0 = pl.BlockSpec((1, RBW, 128), lambda i: (0, i, 0))
    part1 = pl.BlockSpec((1, RBW, 128), lambda i: (1, i, 0))
    return pl.pallas_call(
        _out_body,
        grid=(WB,),
        in_specs=[pl.BlockSpec((5, 128, 128), lambda i: (0, 0, 0)),
                  pl.BlockSpec((1, 128), lambda i: (0, 0)),
                  pl.BlockSpec((8, 128, 128), lambda i: (0, 0, 0)),
                  pl.BlockSpec((8, RBW // 8, RBW), lambda i: (0, 0, 0)),
                  rows, part0, part1, part0, part1],
        out_specs=pl.BlockSpec((RBW // 8, 128), lambda i: (i, 0)),
        out_shape=jax.ShapeDtypeStruct((CR, 128), jnp.float32),
    )(bm, cc, mf, pj, tab_w, p1w, p1w, p2w, p2w)


def kernel(x, edge_index, W_x, b_x, W_y, b_y, W_th, b_th, W_v, b_v,
           W_e1, b_e1, W_x2, b_x2, W_e2, b_e2, W_out, b_out):
    f32 = jnp.float32

    M = jnp.concatenate(
        [W_x @ W_v[0:64], W_y @ W_v[64:128], W_th @ W_v[128:192]], axis=0)
    m0 = b_x @ W_v[0:64] + b_y @ W_v[64:128] + b_th @ W_v[128:192] + b_v
    A = W_e1[0:64]
    B = W_e1[64:128]
    C = W_e1[128:130]
    D = W_e1[130:132]
    p = (W_e2[0:64] @ W_out)[:, 0]
    q = (W_e2[64:128] @ W_out)[:, 0]
    rp = W_x2 @ p
    rq = W_x2 @ q

    def fold(r):
        ar = A @ r
        br = B @ r
        zeros5 = jnp.zeros((5,), f32)
        dst_c = jnp.concatenate([M @ ar, C @ r, (m0 @ ar + b_e1 @ r)[None], zeros5])
        src_c = jnp.concatenate([M @ br, D @ r, (m0 @ br)[None], zeros5])
        return dst_c, src_c

    a1e, a4e = fold(rp)
    b1e, b4e = fold(rq)
    c0 = b_x2 @ p + (b_e2 @ W_out)[0]
    c1 = b_x2 @ q
    bo = b_out[0]

    ones16 = jnp.zeros((16,), f32).at[10].set(1.0)
    cf8 = jnp.stack([b4e, a4e, ones16, b1e, a1e,
                     jnp.zeros(16, f32), jnp.zeros(16, f32),
                     jnp.zeros(16, f32)], axis=1)
    a_idx = jnp.arange(256)
    b_idx = jnp.arange(128)
    same_node = (a_idx[:, None] // 16) == (b_idx[None, :] // F)
    t2 = jnp.where(same_node, cf8[a_idx % 16][:, b_idx % F], 0.0)

    same_grp = (b_idx[:, None] // F) == (b_idx[None, :] // F)
    bms = jnp.stack([
        jnp.where(same_grp & ((b_idx[:, None] % F) == f), 1.0, 0.0)
        for f in (2, 0, 3, 1, 4)], axis=0)
    cc = jnp.zeros((1, 128), f32).at[0, 0].set(c1).at[0, 1].set(c0) \
        .at[0, 2].set(bo)

    mf = jnp.stack([
        jnp.where((b_idx[None, :] // 16 == j)
                  & (b_idx[:, None] == 8 * (b_idx[None, :] % 16)), 1.0, 0.0)
        for j in range(8)], axis=0)
    r_idx = jnp.arange(RBW)
    c_idx = jnp.arange(RBW // 8)
    pj = jnp.stack([
        jnp.where(r_idx[None, :] == 8 * c_idx[:, None] + j, 1.0, 0.0)
        for j in range(8)], axis=0)

    edges = edge_index.astype(jnp.int32).reshape(2, E_BLOCKS, EB)
    zrow = jnp.zeros((400, F), f32)

    xa16 = jnp.pad(jnp.concatenate([x, jnp.ones((N, 1), f32)], axis=1),
                   ((0, N_PAD - N), (0, 5)))
    xw2 = jnp.reshape(xa16, (WR, 256))
    tab_w = _tab_stage(t2, xw2)
    tab8 = jnp.reshape(tab_w, (N_PAD, F))

    p1 = _seg8(tab8, edges, zrow)
    p1w = jnp.reshape(p1, (NC, WR, 128))

    u_w = _u_stage(bms, cc, tab_w, p1w)
    u8 = jnp.reshape(u_w, (N_PAD, F))

    p2 = _seg8(u8, edges, zrow)
    p2w = jnp.reshape(p2, (NC, WR, 128))

    o_c = _out_stage(bms, cc, mf, pj, tab_w, p1w, p2w)
    return jnp.reshape(o_c, (1, N_PAD))[:, :N]
